# baseline (speedup 1.0000x reference)
import functools
import math

import jax
import jax.numpy as jnp
from jax import lax
from jax.experimental import pallas as pl
from jax.experimental.pallas import tpu as pltpu
from jax.experimental.pallas import tpu_sc as plsc

VOCAB = 100000
EMB = 128
SCALE = math.sqrt(EMB)

NC = 2
NS = 16
NW = NC * NS

CH = 128
NBUF = 4
BB = 8


def _make_gather(n_total):
    assert n_total % (NW * CH) == 0
    b_per_w = n_total // NW
    nchunk = b_per_w // CH
    mesh = plsc.VectorSubcoreMesh(core_axis_name="c", subcore_axis_name="s")

    @functools.partial(
        pl.kernel,
        out_type=jax.ShapeDtypeStruct((n_total, EMB), jnp.float32),
        mesh=mesh,
        scratch_types=(
            [pltpu.VMEM((nchunk, CH), jnp.int32)]
            + [pltpu.VMEM((CH, EMB), jnp.float32) for _ in range(NBUF)]
            + [pltpu.SemaphoreType.DMA for _ in range(2 * NBUF)]
        ),
    )
    def gather(tok_hbm, table_hbm, out_hbm, idx_v, *rest):
        bufs = rest[:NBUF]
        gsems = rest[NBUF:2 * NBUF]
        osems = rest[2 * NBUF:]
        wid = lax.axis_index("s") * NC + lax.axis_index("c")
        base = wid * b_per_w

        pltpu.sync_copy(tok_hbm.at[wid], idx_v)

        def start_g(j, b):
            return pltpu.async_copy(table_hbm.at[idx_v.at[j]], bufs[b],
                                    gsems[b])

        def start_o(j, b):
            return pltpu.async_copy(
                bufs[b], out_hbm.at[pl.ds(base + j * CH, CH)], osems[b])

        g_cp = [None] * NBUF
        o_cp = [None] * NBUF
        for j in range(NBUF):
            g_cp[j] = start_g(j, j)
        for j in range(nchunk):
            b = j % NBUF
            m = j + NBUF // 2
            if NBUF <= m < nchunk:
                s = m % NBUF
                o_cp[s].wait()
                g_cp[s] = start_g(m, s)
            g_cp[b].wait()
            o_cp[b] = start_o(j, b)
        for j in range(nchunk - NBUF, nchunk):
            o_cp[j % NBUF].wait()

    return gather


def _scale_fold(flat, bsz, seq):
    def fold_body(t_ref, o_ref):
        for b in range(BB):
            o_ref[b] = t_ref[pl.ds(b * seq, seq), :] * SCALE

    return pl.pallas_call(
        fold_body,
        out_shape=jax.ShapeDtypeStruct((bsz, seq, EMB), jnp.float32),
        grid=(bsz // BB,),
        in_specs=[pl.BlockSpec((BB * seq, EMB), lambda i: (i, 0))],
        out_specs=pl.BlockSpec((BB, seq, EMB), lambda i: (i, 0, 0)),
    )(flat)


def kernel(tokens, table):
    bsz, seq = tokens.shape
    n_total = bsz * seq
    tok = tokens.astype(jnp.int32).reshape(NW, n_total // (NW * CH), CH)
    flat = _make_gather(n_total)(tok, table)
    return _scale_fold(flat, bsz, seq)


# speedup vs baseline: 2.7889x; 2.7889x over previous
import functools
import math

import jax
import jax.numpy as jnp
from jax import lax
from jax.experimental import pallas as pl
from jax.experimental.pallas import tpu as pltpu
from jax.experimental.pallas import tpu_sc as plsc

VOCAB = 100000
EMB = 128
SCALE = math.sqrt(EMB)

NC = 2
NS = 16
NW = NC * NS

CH = 128
NBUF = 4


def _scale_body(t_ref, o_ref):
    o_ref[...] = t_ref[...] * SCALE


def _scale_table(table):
    v, d = table.shape
    blk = 1000
    return pl.pallas_call(
        _scale_body,
        out_shape=jax.ShapeDtypeStruct((v, d), jnp.float32),
        grid=(v // blk,),
        in_specs=[pl.BlockSpec((blk, d), lambda i: (i, 0))],
        out_specs=pl.BlockSpec((blk, d), lambda i: (i, 0)),
    )(table)


def _make_gather(n_total):
    assert n_total % (NW * CH) == 0
    b_per_w = n_total // NW
    nchunk = b_per_w // CH
    mesh = plsc.VectorSubcoreMesh(core_axis_name="c", subcore_axis_name="s")

    @functools.partial(
        pl.kernel,
        out_type=jax.ShapeDtypeStruct((n_total, EMB), jnp.float32),
        mesh=mesh,
        scratch_types=(
            [pltpu.VMEM((nchunk, CH), jnp.int32)]
            + [pltpu.VMEM((CH, EMB), jnp.float32) for _ in range(NBUF)]
            + [pltpu.SemaphoreType.DMA for _ in range(2 * NBUF)]
        ),
    )
    def gather(tok_hbm, table_hbm, out_hbm, idx_v, *rest):
        bufs = rest[:NBUF]
        gsems = rest[NBUF:2 * NBUF]
        osems = rest[2 * NBUF:]
        wid = lax.axis_index("s") * NC + lax.axis_index("c")
        base = wid * b_per_w

        pltpu.sync_copy(tok_hbm.at[wid], idx_v)

        def start_g(j, b):
            return pltpu.async_copy(table_hbm.at[idx_v.at[j]], bufs[b],
                                    gsems[b])

        def start_o(j, b):
            return pltpu.async_copy(
                bufs[b], out_hbm.at[pl.ds(base + j * CH, CH)], osems[b])

        g_cp = [None] * NBUF
        o_cp = [None] * NBUF
        for j in range(NBUF):
            g_cp[j] = start_g(j, j)
        for j in range(nchunk):
            b = j % NBUF
            m = j + NBUF // 2
            if NBUF <= m < nchunk:
                s = m % NBUF
                o_cp[s].wait()
                g_cp[s] = start_g(m, s)
            g_cp[b].wait()
            o_cp[b] = start_o(j, b)
        for j in range(nchunk - NBUF, nchunk):
            o_cp[j % NBUF].wait()

    return gather


def kernel(tokens, table):
    bsz, seq = tokens.shape
    n_total = bsz * seq
    tok = tokens.astype(jnp.int32).T.reshape(NW, n_total // (NW * CH), CH)
    table_scaled = _scale_table(table)
    flat = _make_gather(n_total)(tok, table_scaled)
    return flat.reshape(seq, bsz, EMB).transpose(1, 0, 2)


# speedup vs baseline: 3.5443x; 1.2709x over previous
import functools
import math

import jax
import jax.numpy as jnp
from jax import lax
from jax.experimental import pallas as pl
from jax.experimental.pallas import tpu as pltpu
from jax.experimental.pallas import tpu_sc as plsc

VOCAB = 100000
EMB = 128
SCALE = math.sqrt(EMB)

NC = 2
NS = 16
NW = NC * NS

CH = 128
NBUF = 4


def _scale_body(t_ref, o_ref):
    o_ref[...] = t_ref[...] * SCALE


def _scale_table(table):
    v, d = table.shape
    blk = 4000
    return pl.pallas_call(
        _scale_body,
        out_shape=jax.ShapeDtypeStruct((v, d), jnp.float32),
        grid=(v // blk,),
        in_specs=[pl.BlockSpec((blk, d), lambda i: (i, 0))],
        out_specs=pl.BlockSpec((blk, d), lambda i: (i, 0)),
    )(table)


def _make_gather(n_total):
    assert n_total % (NW * CH) == 0
    b_per_w = n_total // NW
    nchunk = b_per_w // CH
    mesh = plsc.VectorSubcoreMesh(core_axis_name="c", subcore_axis_name="s")

    @functools.partial(
        pl.kernel,
        out_type=jax.ShapeDtypeStruct((n_total, EMB), jnp.float32),
        mesh=mesh,
        scratch_types=(
            [pltpu.VMEM((nchunk, CH), jnp.int32)]
            + [pltpu.VMEM((CH, EMB), jnp.float32) for _ in range(NBUF)]
            + [pltpu.SemaphoreType.DMA for _ in range(2 * NBUF)]
        ),
    )
    def gather(tok_hbm, table_hbm, out_hbm, idx_v, *rest):
        bufs = rest[:NBUF]
        gsems = rest[NBUF:2 * NBUF]
        osems = rest[2 * NBUF:]
        wid = lax.axis_index("s") * NC + lax.axis_index("c")
        base = wid * b_per_w

        pltpu.sync_copy(tok_hbm.at[wid], idx_v)

        def start_g(j, b):
            return pltpu.async_copy(table_hbm.at[idx_v.at[j]], bufs[b],
                                    gsems[b])

        def start_o(j, b):
            return pltpu.async_copy(
                bufs[b], out_hbm.at[pl.ds(base + j * CH, CH)], osems[b])

        g_cp = [None] * NBUF
        o_cp = [None] * NBUF
        for j in range(NBUF):
            g_cp[j] = start_g(j, j)
        for j in range(nchunk):
            b = j % NBUF
            m = j + NBUF // 2
            if NBUF <= m < nchunk:
                s = m % NBUF
                o_cp[s].wait()
                g_cp[s] = start_g(m, s)
            g_cp[b].wait()
            o_cp[b] = start_o(j, b)
        for j in range(nchunk - NBUF, nchunk):
            o_cp[j % NBUF].wait()

    return gather


def kernel(tokens, table):
    bsz, seq = tokens.shape
    n_total = bsz * seq
    tok = tokens.astype(jnp.int32).T.reshape(NW, n_total // (NW * CH), CH)
    table_scaled = _scale_table(table)
    flat = _make_gather(n_total)(tok, table_scaled)
    return flat.reshape(seq, bsz, EMB).transpose(1, 0, 2)


# speedup vs baseline: 3.6390x; 1.0267x over previous
import functools
import math

import jax
import jax.numpy as jnp
from jax import lax
from jax.experimental import pallas as pl
from jax.experimental.pallas import tpu as pltpu
from jax.experimental.pallas import tpu_sc as plsc

VOCAB = 100000
EMB = 128
SCALE = math.sqrt(EMB)

NC = 2
NS = 16
NW = NC * NS

CH = 128
NBUF = 4


def _scale_body(t_ref, o_ref):
    o_ref[...] = t_ref[...] * SCALE


def _scale_table(table):
    v, d = table.shape
    blk = 10000
    return pl.pallas_call(
        _scale_body,
        out_shape=jax.ShapeDtypeStruct((v, d), jnp.float32),
        grid=(v // blk,),
        in_specs=[pl.BlockSpec((blk, d), lambda i: (i, 0))],
        out_specs=pl.BlockSpec((blk, d), lambda i: (i, 0)),
    )(table)


def _make_gather(n_total):
    assert n_total % (NW * CH) == 0
    b_per_w = n_total // NW
    nchunk = b_per_w // CH
    mesh = plsc.VectorSubcoreMesh(core_axis_name="c", subcore_axis_name="s")

    @functools.partial(
        pl.kernel,
        out_type=jax.ShapeDtypeStruct((n_total, EMB), jnp.float32),
        mesh=mesh,
        scratch_types=(
            [pltpu.VMEM((nchunk, CH), jnp.int32)]
            + [pltpu.VMEM((CH, EMB), jnp.float32) for _ in range(NBUF)]
            + [pltpu.SemaphoreType.DMA for _ in range(2 * NBUF)]
        ),
    )
    def gather(tok_hbm, table_hbm, out_hbm, idx_v, *rest):
        bufs = rest[:NBUF]
        gsems = rest[NBUF:2 * NBUF]
        osems = rest[2 * NBUF:]
        wid = lax.axis_index("s") * NC + lax.axis_index("c")
        base = wid * b_per_w

        pltpu.sync_copy(tok_hbm.at[wid], idx_v)

        def start_g(j, b):
            return pltpu.async_copy(table_hbm.at[idx_v.at[j]], bufs[b],
                                    gsems[b])

        def start_o(j, b):
            return pltpu.async_copy(
                bufs[b], out_hbm.at[pl.ds(base + j * CH, CH)], osems[b])

        g_cp = [None] * NBUF
        o_cp = [None] * NBUF
        for j in range(NBUF):
            g_cp[j] = start_g(j, j)
        for j in range(nchunk):
            b = j % NBUF
            m = j + NBUF // 2
            if NBUF <= m < nchunk:
                s = m % NBUF
                o_cp[s].wait()
                g_cp[s] = start_g(m, s)
            g_cp[b].wait()
            o_cp[b] = start_o(j, b)
        for j in range(nchunk - NBUF, nchunk):
            o_cp[j % NBUF].wait()

    return gather


def kernel(tokens, table):
    bsz, seq = tokens.shape
    n_total = bsz * seq
    tok = tokens.astype(jnp.int32).T.reshape(NW, n_total // (NW * CH), CH)
    table_scaled = _scale_table(table)
    flat = _make_gather(n_total)(tok, table_scaled)
    return flat.reshape(seq, bsz, EMB).transpose(1, 0, 2)


# speedup vs baseline: 4.8702x; 1.3383x over previous
import functools
import math

import jax
import jax.numpy as jnp
from jax import lax
from jax.experimental import pallas as pl
from jax.experimental.pallas import tpu as pltpu
from jax.experimental.pallas import tpu_sc as plsc

VOCAB = 100000
EMB = 128
SCALE = math.sqrt(EMB)

NC = 2
NS = 16
NW = NC * NS

CH = 128
NBUF = 5
NSL = EMB // 16


def _make_gather(n_total):
    assert n_total % (NW * CH * NBUF) == 0
    b_per_w = n_total // NW
    nchunk = b_per_w // CH
    ngroup = nchunk // NBUF
    mesh = plsc.VectorSubcoreMesh(core_axis_name="c", subcore_axis_name="s")

    @functools.partial(
        pl.kernel,
        out_type=jax.ShapeDtypeStruct((n_total, EMB), jnp.float32),
        mesh=mesh,
        scratch_types=(
            [pltpu.VMEM((nchunk, CH), jnp.int32)]
            + [pltpu.VMEM((CH, EMB), jnp.float32) for _ in range(NBUF)]
            + [pltpu.SemaphoreType.DMA for _ in range(2 * NBUF)]
        ),
    )
    def gather(tok_hbm, table_hbm, out_hbm, idx_v, *rest):
        bufs = rest[:NBUF]
        gsems = rest[NBUF:2 * NBUF]
        osems = rest[2 * NBUF:]
        wid = lax.axis_index("s") * NC + lax.axis_index("c")
        base = wid * b_per_w

        pltpu.sync_copy(tok_hbm.at[wid], idx_v)

        def start_g(j, b):
            return pltpu.async_copy(table_hbm.at[idx_v.at[j]], bufs[b],
                                    gsems[b])

        def start_o(j, b):
            return pltpu.async_copy(
                bufs[b], out_hbm.at[pl.ds(base + j * CH, CH)], osems[b])

        g_cp = [start_g(b, b) for b in range(NBUF)]
        o_cp = [None] * NBUF

        def group(g, _):
            for b in range(NBUF):
                jj = g * NBUF + b
                g_cp[b].wait()
                buf = bufs[b]

                @plsc.parallel_loop(0, CH, unroll=4)
                def _scale_row(r):
                    for c in range(NSL):
                        sl = (r, pl.ds(c * 16, 16))
                        buf[sl] = buf[sl] * SCALE

                o_cp[b] = start_o(jj, b)
            for b in range(NBUF):
                jn = g * NBUF + NBUF + b
                o_cp[b].wait()

                @pl.when(jn < nchunk)
                def _():
                    start_g(jn, b)
            return 0

        lax.fori_loop(0, ngroup, group, 0, unroll=False)

    return gather


def kernel(tokens, table):
    bsz, seq = tokens.shape
    n_total = bsz * seq
    tok = tokens.astype(jnp.int32).T.reshape(NW, n_total // (NW * CH), CH)
    flat = _make_gather(n_total)(tok, table)
    return flat.reshape(seq, bsz, EMB).transpose(1, 0, 2)


# speedup vs baseline: 4.9261x; 1.0115x over previous
import functools
import math

import jax
import jax.numpy as jnp
from jax import lax
from jax.experimental import pallas as pl
from jax.experimental.pallas import tpu as pltpu
from jax.experimental.pallas import tpu_sc as plsc

VOCAB = 100000
EMB = 128
SCALE = math.sqrt(EMB)

NC = 2
NS = 16
NW = NC * NS

CH = 128
NBUF = 5
NSL = EMB // 16


def _make_gather(n_total):
    assert n_total % (NW * CH * NBUF) == 0
    b_per_w = n_total // NW
    nchunk = b_per_w // CH
    ngroup = nchunk // NBUF
    mesh = plsc.VectorSubcoreMesh(core_axis_name="c", subcore_axis_name="s")

    @functools.partial(
        pl.kernel,
        out_type=jax.ShapeDtypeStruct((n_total, EMB), jnp.float32),
        mesh=mesh,
        scratch_types=(
            [pltpu.VMEM((nchunk, CH), jnp.int32)]
            + [pltpu.VMEM((CH, EMB), jnp.float32) for _ in range(NBUF)]
            + [pltpu.SemaphoreType.DMA for _ in range(2 * NBUF)]
        ),
    )
    def gather(tok_hbm, table_hbm, out_hbm, idx_v, *rest):
        bufs = rest[:NBUF]
        gsems = rest[NBUF:2 * NBUF]
        osems = rest[2 * NBUF:]
        wid = lax.axis_index("s") * NC + lax.axis_index("c")
        base = wid * b_per_w

        pltpu.sync_copy(tok_hbm.at[wid], idx_v)

        def start_g(j, b):
            return pltpu.async_copy(table_hbm.at[idx_v.at[j]], bufs[b],
                                    gsems[b])

        def start_o(j, b):
            return pltpu.async_copy(
                bufs[b], out_hbm.at[pl.ds(base + j * CH, CH)], osems[b])

        g_cp = [start_g(b, b) for b in range(NBUF)]
        o_cp = [None] * NBUF

        def refill(g, p):
            jn = (g + 1) * NBUF + p
            o_cp[p].wait()

            @pl.when(jn < nchunk)
            def _():
                start_g(jn, p)

        def group(g, _):
            for b in range(NBUF):
                jj = g * NBUF + b
                g_cp[b].wait()
                buf = bufs[b]

                @plsc.parallel_loop(0, CH, unroll=4)
                def _scale_row(r):
                    for c in range(NSL):
                        sl = (r, pl.ds(c * 16, 16))
                        buf[sl] = buf[sl] * SCALE

                o_cp[b] = start_o(jj, b)
                if b >= 1:
                    refill(g, b - 1)
            refill(g, NBUF - 1)
            return 0

        lax.fori_loop(0, ngroup, group, 0, unroll=False)

    return gather


def kernel(tokens, table):
    bsz, seq = tokens.shape
    n_total = bsz * seq
    tok = tokens.astype(jnp.int32).T.reshape(NW, n_total // (NW * CH), CH)
    flat = _make_gather(n_total)(tok, table)
    return flat.reshape(seq, bsz, EMB).transpose(1, 0, 2)
